# Initial kernel scaffold; baseline (speedup 1.0000x reference)
import jax
import jax.numpy as jnp
from jax.experimental import pallas as pl


def kernel(x, table1, table2, W, b):
    raise NotImplementedError("write your pallas kernel here")


# speedup vs baseline: 141.3245x; 141.3245x over previous
import functools

import jax
import jax.numpy as jnp
from jax import lax
from jax.experimental import pallas as pl
from jax.experimental.pallas import tpu as pltpu
from jax.experimental.pallas import tpu_sc as plsc

_NC = 2
_NS = 16
_NW = _NC * _NS
_L = 16


def _make_sc_kernel(N: int, C: int):
    n_per_w = N // _NW
    chunks = n_per_w // C
    mesh = plsc.VectorSubcoreMesh(core_axis_name="c", subcore_axis_name="s")

    @functools.partial(
        pl.kernel,
        out_type=jax.ShapeDtypeStruct((2 * N,), jnp.float32),
        mesh=mesh,
        compiler_params=pltpu.CompilerParams(needs_layout_passes=False),
        scratch_types=[
            pltpu.VMEM((C,), jnp.int32),
            pltpu.VMEM((C,), jnp.float32),
            pltpu.VMEM((C,), jnp.float32),
            pltpu.VMEM((_L, _L), jnp.float32),
            pltpu.VMEM((_L, _L), jnp.float32),
            pltpu.VMEM((_L,), jnp.float32),
            pltpu.VMEM((_L,), jnp.float32),
            pltpu.VMEM((_L,), jnp.float32),
            pltpu.VMEM((_L,), jnp.float32),
        ],
    )
    def sc_kernel(x_hbm, t1_hbm, t2_hbm, w_hbm, b_hbm, out_hbm,
                  x_v, o1_v, o2_v, t1_v, t2_v, w_v, b_v, lut1_v, lut2_v):
        pltpu.sync_copy(t1_hbm, t1_v)
        pltpu.sync_copy(t2_hbm, t2_v)
        pltpu.sync_copy(w_hbm, w_v)
        pltpu.sync_copy(b_hbm, b_v)

        lanes = lax.iota(jnp.int32, _L)
        wv = w_v[...]
        acc1 = b_v[...]
        acc2 = b_v[...]
        for k in range(10):
            kk = jnp.full((_L,), k, jnp.int32)
            wk = jnp.sum(jnp.where(lanes == k, wv, 0.0))
            acc1 = acc1 + plsc.load_gather(t1_v, [lanes, kk]) * wk
            acc2 = acc2 + plsc.load_gather(t2_v, [lanes, kk]) * wk
        lut1_v[...] = acc1
        lut2_v[...] = acc2

        wid = lax.axis_index("s") * _NC + lax.axis_index("c")
        base = wid * n_per_w

        for c in range(chunks):
            off = base + c * C
            pltpu.sync_copy(x_hbm.at[pl.ds(off, C)], x_v)

            @plsc.parallel_loop(0, C // _L, unroll=8)
            def _(i):
                s = i * _L
                idx = x_v[pl.ds(s, _L)]
                o1_v[pl.ds(s, _L)] = plsc.load_gather(lut1_v, [idx])
                o2_v[pl.ds(s, _L)] = plsc.load_gather(lut2_v, [idx])

            pltpu.sync_copy(o1_v, out_hbm.at[pl.ds(off, C)])
            pltpu.sync_copy(o2_v, out_hbm.at[pl.ds(N + off, C)])

    return sc_kernel


@jax.jit
def kernel(x, table1, table2, W, b):
    B, F = x.shape
    N = B * F
    xf = x.reshape(N)
    t1p = jnp.zeros((_L, _L), jnp.float32).at[:10, :10].set(table1)
    t2p = jnp.zeros((_L, _L), jnp.float32).at[:10, :10].set(table2)
    wp = jnp.zeros((_L,), jnp.float32).at[:10].set(W[:, 0])
    bp = jnp.broadcast_to(b, (_L,))
    out = _make_sc_kernel(N, C=12800)(xf, t1p, t2p, wp, bp)
    return out.reshape(2 * B, F, 1)


# speedup vs baseline: 201.3716x; 1.4249x over previous
import functools

import jax
import jax.numpy as jnp
from jax import lax
from jax.experimental import pallas as pl
from jax.experimental.pallas import tpu as pltpu
from jax.experimental.pallas import tpu_sc as plsc

_NC = 2
_NS = 16
_NW = _NC * _NS
_L = 16


def _make_sc_kernel(B: int, F: int, R: int):
    rows_per_w = B // _NW
    chunks = rows_per_w // R
    blocks_per_col = R // _L
    mesh = plsc.VectorSubcoreMesh(core_axis_name="c", subcore_axis_name="s")

    @functools.partial(
        pl.kernel,
        out_type=jax.ShapeDtypeStruct((F, 2 * B), jnp.float32),
        mesh=mesh,
        compiler_params=pltpu.CompilerParams(needs_layout_passes=False),
        scratch_types=[
            pltpu.VMEM((R * F,), jnp.int32),
            pltpu.VMEM((F, R), jnp.float32),
            pltpu.VMEM((F, R), jnp.float32),
            pltpu.VMEM((_L, _L), jnp.float32),
            pltpu.VMEM((_L, _L), jnp.float32),
            pltpu.VMEM((_L,), jnp.float32),
            pltpu.VMEM((_L,), jnp.float32),
            pltpu.VMEM((_L,), jnp.float32),
            pltpu.VMEM((_L,), jnp.float32),
        ],
    )
    def sc_kernel(x_hbm, t1_hbm, t2_hbm, w_hbm, b_hbm, out_hbm,
                  x_v, o1_v, o2_v, t1_v, t2_v, w_v, b_v, lut1_v, lut2_v):
        pltpu.sync_copy(t1_hbm, t1_v)
        pltpu.sync_copy(t2_hbm, t2_v)
        pltpu.sync_copy(w_hbm, w_v)
        pltpu.sync_copy(b_hbm, b_v)

        lanes = lax.iota(jnp.int32, _L)
        wv = w_v[...]
        acc1 = b_v[...]
        acc2 = b_v[...]
        for k in range(10):
            kk = jnp.full((_L,), k, jnp.int32)
            wk = jnp.sum(jnp.where(lanes == k, wv, 0.0))
            acc1 = acc1 + plsc.load_gather(t1_v, [lanes, kk]) * wk
            acc2 = acc2 + plsc.load_gather(t2_v, [lanes, kk]) * wk
        lut1_v[...] = acc1
        lut2_v[...] = acc2

        wid = lax.axis_index("s") * _NC + lax.axis_index("c")
        row0 = wid * rows_per_w
        lanesF = lanes * F

        for c in range(chunks):
            r0 = row0 + c * R
            pltpu.sync_copy(x_hbm.at[pl.ds(r0 * F, R * F)], x_v)

            @plsc.parallel_loop(0, F * blocks_per_col, unroll=8)
            def _(t):
                f = t >> 4
                rb = (t & (blocks_per_col - 1)) * _L
                idx = plsc.load_gather(x_v, [lanesF + (rb * F + f)])
                o1_v[f, pl.ds(rb, _L)] = plsc.load_gather(lut1_v, [idx])
                o2_v[f, pl.ds(rb, _L)] = plsc.load_gather(lut2_v, [idx])

            pltpu.sync_copy(o1_v, out_hbm.at[:, pl.ds(r0, R)])
            pltpu.sync_copy(o2_v, out_hbm.at[:, pl.ds(B + r0, R)])

    return sc_kernel


@jax.jit
def kernel(x, table1, table2, W, b):
    B, F = x.shape
    xf = x.reshape(B * F)
    t1p = jnp.zeros((_L, _L), jnp.float32).at[:10, :10].set(table1)
    t2p = jnp.zeros((_L, _L), jnp.float32).at[:10, :10].set(table2)
    wp = jnp.zeros((_L,), jnp.float32).at[:10].set(W[:, 0])
    bp = jnp.broadcast_to(b, (_L,))
    out_fm = _make_sc_kernel(B, F, R=256)(xf, t1p, t2p, wp, bp)
    return out_fm.T.reshape(2 * B, F, 1)
